# baseline (speedup 1.0000x reference)
import functools

import jax
import jax.numpy as jnp
from jax import lax
from jax.experimental import pallas as pl
from jax.experimental.pallas import tpu as pltpu
from jax.experimental.pallas import tpu_sc as plsc

B = 16384
D = 32
NC = 2
NS = 16
NW = NC * NS
BPW = B // NW
PASS = 128
NPASS = BPW // PASS


def _sc_body(uidx_hbm, iidx_hbm, uemb_hbm, iemb_hbm, ub_hbm, ib_hbm,
             partials_hbm, bias_hbm,
             uidx_v, iidx_v, u_dst, v_dst, ub_dst, ib_dst, acc_v, outb_v,
             sem_u, sem_v, sem_ub, sem_ib):
    wid = lax.axis_index("s") * NC + lax.axis_index("c")
    base = wid * BPW
    row0 = wid * (BPW // 128)

    pltpu.sync_copy(uidx_hbm.at[pl.ds(row0, BPW // 128)], uidx_v)
    pltpu.sync_copy(iidx_hbm.at[pl.ds(row0, BPW // 128)], iidx_v)

    lanes = lax.iota(jnp.int32, 16)
    zeros16 = jnp.zeros((16,), jnp.int32)
    acc = jnp.zeros((16,), jnp.float32)

    for p in range(NPASS):
        def enqueue(c, carry):
            uvec = uidx_v[p, pl.ds(c * 16, 16)]
            ivec = iidx_v[p, pl.ds(c * 16, 16)]
            n0 = c * 16
            for k in range(16):
                su = uvec[k]
                si = ivec[k]
                pltpu.async_copy(
                    uemb_hbm.at[pl.ds(su, 1), :],
                    u_dst.at[pl.ds(n0 + k, 1), :], sem_u)
                pltpu.async_copy(
                    iemb_hbm.at[pl.ds(si, 1), :],
                    v_dst.at[pl.ds(n0 + k, 1), :], sem_v)
                pltpu.async_copy(
                    ub_hbm.at[pl.ds(su, 1), :],
                    ub_dst.at[pl.ds(n0 + k, 1), :], sem_ub)
                pltpu.async_copy(
                    ib_hbm.at[pl.ds(si, 1), :],
                    ib_dst.at[pl.ds(n0 + k, 1), :], sem_ib)
            return carry

        lax.fori_loop(0, PASS // 16, enqueue, 0)

        pltpu.make_async_copy(
            uemb_hbm.at[pl.ds(0, PASS), :], u_dst, sem_u).wait()
        pltpu.make_async_copy(
            iemb_hbm.at[pl.ds(0, PASS), :], v_dst, sem_v).wait()
        pltpu.make_async_copy(
            ub_hbm.at[pl.ds(0, PASS), :], ub_dst, sem_ub).wait()
        pltpu.make_async_copy(
            ib_hbm.at[pl.ds(0, PASS), :], ib_dst, sem_ib).wait()

        def dot_body(n, a):
            u0 = u_dst[n, pl.ds(0, 16)]
            u1 = u_dst[n, pl.ds(16, 16)]
            w0 = v_dst[n, pl.ds(0, 16)]
            w1 = v_dst[n, pl.ds(16, 16)]
            return a + u0 * w0 + u1 * w1

        acc = lax.fori_loop(0, PASS, dot_body, acc, unroll=4)

        for j in range(PASS // 16):
            rows = lanes + j * 16
            ubg = plsc.load_gather(ub_dst, [rows, zeros16])
            ibg = plsc.load_gather(ib_dst, [rows, zeros16])
            outb_v[pl.ds(p * PASS + j * 16, 16)] = ubg + ibg

    acc_v[...] = acc
    pltpu.sync_copy(acc_v, partials_hbm.at[wid])
    pltpu.sync_copy(outb_v, bias_hbm.at[pl.ds(base, BPW)])


@functools.cache
def _make_sc_call():
    return pl.kernel(
        _sc_body,
        out_type=[
            jax.ShapeDtypeStruct((NW, 16), jnp.float32),
            jax.ShapeDtypeStruct((B,), jnp.float32),
        ],
        mesh=plsc.VectorSubcoreMesh(
            core_axis_name="c", subcore_axis_name="s"),
        compiler_params=pltpu.CompilerParams(
            use_tc_tiling_on_sc=True, needs_layout_passes=False),
        scratch_types=[
            pltpu.VMEM((BPW // 128, 128), jnp.int32),
            pltpu.VMEM((BPW // 128, 128), jnp.int32),
            pltpu.VMEM((PASS, D), jnp.float32),
            pltpu.VMEM((PASS, D), jnp.float32),
            pltpu.VMEM((PASS, 1), jnp.float32),
            pltpu.VMEM((PASS, 1), jnp.float32),
            pltpu.VMEM((16,), jnp.float32),
            pltpu.VMEM((BPW,), jnp.float32),
            pltpu.SemaphoreType.DMA,
            pltpu.SemaphoreType.DMA,
            pltpu.SemaphoreType.DMA,
            pltpu.SemaphoreType.DMA,
        ],
    )


def _tc_body(bias_ref, partials_ref, out_ref):
    total = jnp.sum(partials_ref[...])
    out_ref[...] = bias_ref[...] + total


_tc_call = pl.pallas_call(
    _tc_body,
    out_shape=jax.ShapeDtypeStruct((128, 128), jnp.float32),
)


def kernel(user_index, item_index, user_emb, item_emb, ub, ib):
    uidx2d = user_index.astype(jnp.int32).reshape(B // 128, 128)
    iidx2d = item_index.astype(jnp.int32).reshape(B // 128, 128)
    partials, bias = _make_sc_call()(uidx2d, iidx2d, user_emb, item_emb,
                                     ub, ib)
    out2d = _tc_call(bias.reshape(128, 128), partials)
    return out2d.reshape(B)
